# baseline (speedup 1.0000x reference)
import functools

import jax
import jax.numpy as jnp
from jax import lax
from jax.experimental import pallas as pl
from jax.experimental.pallas import tpu as pltpu
from jax.experimental.pallas import tpu_sc as plsc

_N = 10000
_E = 320000
_F = 128
_G = 16
_EPS = 1e-5

_NC = 2
_NS = 16
_NW = _NC * _NS
_CH = 64
_NCHUNK = 160
_EPT = _NCHUNK * _CH
_EPAD = _EPT * _NW
_NPAD = 10112
_STRIPE = _NPAD // _NS

_sc_mesh = plsc.VectorSubcoreMesh(core_axis_name="c", subcore_axis_name="s",
                                  num_cores=_NC, num_subcores=_NS)
_sc_params = pltpu.CompilerParams(needs_layout_passes=False)


@functools.partial(
    pl.kernel,
    out_type=jax.ShapeDtypeStruct((_NW, _NPAD), jnp.float32),
    mesh=_sc_mesh,
    scratch_types=[pltpu.VMEM((_NCHUNK, _CH), jnp.int32),
                   pltpu.VMEM((_NPAD,), jnp.float32)],
    compiler_params=_sc_params)
def _sc_degree(dst_hbm, out_hbm, dst_v, deg_v):
    cid = lax.axis_index("c")
    sid = lax.axis_index("s")
    wid = sid * _NC + cid
    pltpu.sync_copy(dst_hbm.at[wid], dst_v)
    zeros = jnp.zeros((16,), jnp.float32)

    @pl.loop(0, _NPAD, step=16)
    def _(i):
        deg_v[pl.ds(i, 16)] = zeros

    ones = jnp.ones((16,), jnp.float32)

    @pl.loop(0, _NCHUNK)
    def _(j):
        for k in range(_CH // 16):
            idx = dst_v[j, pl.ds(k * 16, 16)]
            plsc.addupdate_scatter(deg_v, [idx], ones)

    pltpu.sync_copy(deg_v, out_hbm.at[wid])


_NBUF = 4
_NSTAGE = 4
_NHALF = _NCHUNK // _NSTAGE


@functools.partial(
    pl.kernel,
    out_type=jax.ShapeDtypeStruct((_NC, _NPAD, _F), jnp.float32),
    mesh=_sc_mesh,
    scratch_types=[pltpu.VMEM((_NHALF, _CH), jnp.int32),
                   pltpu.VMEM((_NHALF, _CH), jnp.int32),
                   pltpu.VMEM((_CH, _F), jnp.float32),
                   pltpu.VMEM((_CH, _F), jnp.float32),
                   pltpu.VMEM((_CH, _F), jnp.float32),
                   pltpu.VMEM((_CH, _F), jnp.float32),
                   pltpu.SemaphoreType.DMA,
                   pltpu.SemaphoreType.DMA,
                   pltpu.SemaphoreType.DMA,
                   pltpu.SemaphoreType.DMA,
                   pltpu.SemaphoreType.DMA,
                   pltpu.SemaphoreType.DMA,
                   pltpu.SemaphoreType.DMA,
                   pltpu.SemaphoreType.DMA,
                   pltpu.VMEM_SHARED((_NPAD, _F), jnp.float32)],
    compiler_params=_sc_params)
def _sc_scatter(hs_hbm, src_hbm, dst_hbm, zeros_hbm, out_hbm,
                src_v, dst_v, b0, b1, b2, b3,
                g0, g1, g2, g3, t0, t1, t2, t3, acc_sh):
    bufs = (b0, b1, b2, b3)
    gsem = (g0, g1, g2, g3)
    ssem = (t0, t1, t2, t3)
    cid = lax.axis_index("c")
    sid = lax.axis_index("s")
    wid = sid * _NC + cid
    row0 = sid * _STRIPE

    def wait_gather(b):
        pltpu.make_async_copy(hs_hbm.at[src_v.at[0]], bufs[b],
                              gsem[b]).wait()

    def wait_scatter(b):
        pltpu.make_async_copy(bufs[b], acc_sh.at[dst_v.at[0]],
                              ssem[b]).wait()

    def visit(jj, b, scatwait, prefetch):
        b2 = (b + 2) % _NBUF
        if scatwait:
            wait_scatter(b2)
        if prefetch:
            pltpu.async_copy(hs_hbm.at[src_v.at[jj + 2]], bufs[b2],
                             gsem[b2])
        wait_gather(b)
        pltpu.async_copy(bufs[b], acc_sh.at[dst_v.at[jj]], ssem[b],
                         add=True)

    zeroed = False
    for half in range(_NSTAGE):
        base = half * _NHALF
        pltpu.sync_copy(src_hbm.at[wid].at[pl.ds(base, _NHALF)], src_v)
        pltpu.sync_copy(dst_hbm.at[wid].at[pl.ds(base, _NHALF)], dst_v)
        pltpu.async_copy(hs_hbm.at[src_v.at[0]], bufs[0], gsem[0])
        pltpu.async_copy(hs_hbm.at[src_v.at[1]], bufs[1], gsem[1])
        if not zeroed:
            pltpu.sync_copy(zeros_hbm.at[pl.ds(row0, _STRIPE)],
                            acc_sh.at[pl.ds(row0, _STRIPE)])
            plsc.subcore_barrier()
            zeroed = True

        visit(0, 0, scatwait=False, prefetch=True)
        visit(1, 1, scatwait=False, prefetch=True)

        @pl.loop(2, _NHALF - 2, step=_NBUF)
        def _(j):
            for k in range(_NBUF):
                visit(j + k, (k + 2) % _NBUF, scatwait=True, prefetch=True)

        visit(_NHALF - 2, 2, scatwait=True, prefetch=False)
        visit(_NHALF - 1, 3, scatwait=True, prefetch=False)
        wait_scatter(2)
        wait_scatter(3)

    plsc.subcore_barrier()
    pltpu.sync_copy(acc_sh.at[pl.ds(row0, _STRIPE)],
                    out_hbm.at[cid].at[pl.ds(row0, _STRIPE)])


def _tc_pre_body(degp_ref, x_ref, w_ref, dinv_ref, hs_ref):
    deg = jnp.sum(degp_ref[...], axis=0).reshape(_NPAD, 1) + 1.0
    rows = lax.broadcasted_iota(jnp.int32, (_NPAD, 1), 0)
    dinv = jnp.where(rows < _N, lax.rsqrt(deg), 0.0)
    dinv_ref[...] = dinv
    h = jnp.dot(x_ref[...], w_ref[...], preferred_element_type=jnp.float32)
    hs_ref[0:_N, :] = dinv[0:_N, :] * h
    hs_ref[_N:_NPAD, :] = jnp.zeros((_NPAD - _N, _F), jnp.float32)


_tc_pre = pl.pallas_call(
    _tc_pre_body,
    out_shape=(jax.ShapeDtypeStruct((_NPAD, 1), jnp.float32),
               jax.ShapeDtypeStruct((_NPAD, _F), jnp.float32)))


def _tc_mid_body(p_ref, hs_ref, dinv_ref, b_ref, g_ref, be_ref, w_ref,
                 out_ref):
    dinv = dinv_ref[0:_N, :]
    agg = p_ref[0, 0:_N, :] + p_ref[1, 0:_N, :] + hs_ref[0:_N, :]
    pre = dinv * agg + b_ref[...]
    act = jnp.where(pre > 0, pre, 0.01 * pre)
    mu = jnp.mean(act, axis=0, keepdims=True)
    cen = act - mu
    var = jnp.mean(cen * cen, axis=0, keepdims=True)
    bn = cen * (g_ref[...] * lax.rsqrt(var + _EPS)) + be_ref[...]
    h = jnp.dot(bn, w_ref[...], preferred_element_type=jnp.float32)
    out_ref[0:_N, :] = dinv * h
    out_ref[_N:_NPAD, :] = jnp.zeros((_NPAD - _N, _F), jnp.float32)


_tc_mid = pl.pallas_call(
    _tc_mid_body,
    out_shape=jax.ShapeDtypeStruct((_NPAD, _F), jnp.float32))


def _tc_fin_body(p_ref, hs_ref, dinv_ref, b_ref, g_ref, be_ref, batch_ref,
                 out_ref):
    dinv = dinv_ref[0:_N, :]
    agg = p_ref[0, 0:_N, :] + p_ref[1, 0:_N, :] + hs_ref[0:_N, :]
    pre = dinv * agg + b_ref[...]
    act = jnp.where(pre > 0, pre, 0.01 * pre)
    mu = jnp.mean(act, axis=0, keepdims=True)
    cen = act - mu
    var = jnp.mean(cen * cen, axis=0, keepdims=True)
    bn = cen * (g_ref[...] * lax.rsqrt(var + _EPS)) + be_ref[...]
    seg = lax.broadcasted_iota(jnp.int32, (_G, _N), 0)
    onehot = (batch_ref[...].reshape(1, _N) == seg).astype(jnp.float32)
    sums = jnp.dot(onehot, bn, preferred_element_type=jnp.float32)
    cnt = jnp.sum(onehot, axis=1, keepdims=True)
    out_ref[...] = sums / jnp.maximum(cnt, 1.0)


_tc_fin = pl.pallas_call(
    _tc_fin_body,
    out_shape=jax.ShapeDtypeStruct((_G, _F), jnp.float32))


def kernel(x, W1, b1, g1, be1, W2, b2, g2, be2, W3, b3, g3, be3,
           edge_index, batch):
    src = edge_index[0].astype(jnp.int32)
    dst = edge_index[1].astype(jnp.int32)
    perw = _E // _NW
    padw = _EPT - perw
    pad = _N + (jnp.arange(_NW * padw, dtype=jnp.int32) % (_NPAD - _N))
    pad = pad.reshape(_NW, padw)
    srcp = jnp.concatenate([src.reshape(_NW, perw), pad],
                           axis=1).reshape(_NW, _NCHUNK, _CH)
    dstp = jnp.concatenate([dst.reshape(_NW, perw), pad],
                           axis=1).reshape(_NW, _NCHUNK, _CH)
    zeros = jnp.zeros((_NPAD, _F), jnp.float32)

    batchp = batch.astype(jnp.int32).reshape(1, _N)

    degp = _sc_degree(dstp)
    dinv, hs = _tc_pre(degp, x, W1)
    p = _sc_scatter(hs, srcp, dstp, zeros)
    hs = _tc_mid(p, hs, dinv, b1.reshape(1, _F), g1.reshape(1, _F),
                 be1.reshape(1, _F), W2)
    p = _sc_scatter(hs, srcp, dstp, zeros)
    hs = _tc_mid(p, hs, dinv, b2.reshape(1, _F), g2.reshape(1, _F),
                 be2.reshape(1, _F), W3)
    p = _sc_scatter(hs, srcp, dstp, zeros)
    return _tc_fin(p, hs, dinv, b3.reshape(1, _F), g3.reshape(1, _F),
                   be3.reshape(1, _F), batchp)


# speedup vs baseline: 1.0273x; 1.0273x over previous
import functools

import jax
import jax.numpy as jnp
from jax import lax
from jax.experimental import pallas as pl
from jax.experimental.pallas import tpu as pltpu
from jax.experimental.pallas import tpu_sc as plsc

_N = 10000
_E = 320000
_F = 128
_G = 16
_EPS = 1e-5

_NC = 2
_NS = 16
_NW = _NC * _NS
_CH = 128
_NCHUNK = 80
_EPT = _NCHUNK * _CH
_EPAD = _EPT * _NW
_NPAD = 10112
_STRIPE = _NPAD // _NS

_sc_mesh = plsc.VectorSubcoreMesh(core_axis_name="c", subcore_axis_name="s",
                                  num_cores=_NC, num_subcores=_NS)
_sc_params = pltpu.CompilerParams(needs_layout_passes=False)


@functools.partial(
    pl.kernel,
    out_type=jax.ShapeDtypeStruct((_NW, _NPAD), jnp.float32),
    mesh=_sc_mesh,
    scratch_types=[pltpu.VMEM((_NCHUNK, _CH), jnp.int32),
                   pltpu.VMEM((_NPAD,), jnp.float32)],
    compiler_params=_sc_params)
def _sc_degree(dst_hbm, out_hbm, dst_v, deg_v):
    cid = lax.axis_index("c")
    sid = lax.axis_index("s")
    wid = sid * _NC + cid
    pltpu.sync_copy(dst_hbm.at[wid], dst_v)
    zeros = jnp.zeros((16,), jnp.float32)

    @pl.loop(0, _NPAD, step=16)
    def _(i):
        deg_v[pl.ds(i, 16)] = zeros

    ones = jnp.ones((16,), jnp.float32)

    @pl.loop(0, _NCHUNK)
    def _(j):
        for k in range(_CH // 16):
            idx = dst_v[j, pl.ds(k * 16, 16)]
            plsc.addupdate_scatter(deg_v, [idx], ones)

    pltpu.sync_copy(deg_v, out_hbm.at[wid])


_NBUF = 2
_NHALF = _NCHUNK // 2


@functools.partial(
    pl.kernel,
    out_type=jax.ShapeDtypeStruct((_NC, _NPAD, _F), jnp.float32),
    mesh=_sc_mesh,
    scratch_types=[pltpu.VMEM((_NHALF, _CH), jnp.int32),
                   pltpu.VMEM((_NHALF, _CH), jnp.int32),
                   pltpu.VMEM((_CH, _F), jnp.float32),
                   pltpu.VMEM((_CH, _F), jnp.float32),
                   pltpu.SemaphoreType.DMA,
                   pltpu.SemaphoreType.DMA,
                   pltpu.VMEM_SHARED((_NPAD, _F), jnp.float32)],
    compiler_params=_sc_params)
def _sc_scatter(hs_hbm, src_hbm, dst_hbm, zeros_hbm, out_hbm,
                src_v, dst_v, b0, b1, s0, s1, acc_sh):
    bufs = (b0, b1)
    sems = (s0, s1)
    cid = lax.axis_index("c")
    sid = lax.axis_index("s")
    wid = sid * _NC + cid
    row0 = sid * _STRIPE
    zeroed = False

    for half in range(2):
        base = half * _NHALF
        pltpu.sync_copy(src_hbm.at[wid].at[pl.ds(base, _NHALF)], src_v)
        pltpu.sync_copy(dst_hbm.at[wid].at[pl.ds(base, _NHALF)], dst_v)
        for b in range(_NBUF):
            pltpu.async_copy(hs_hbm.at[src_v.at[b]], bufs[b], sems[b])
        if not zeroed:
            pltpu.sync_copy(zeros_hbm.at[pl.ds(row0, _STRIPE)],
                            acc_sh.at[pl.ds(row0, _STRIPE)])
            plsc.subcore_barrier()
            zeroed = True

        @pl.loop(0, _NHALF - _NBUF, step=_NBUF)
        def _(j):
            for b in range(_NBUF):
                jj = j + b
                pltpu.make_async_copy(hs_hbm.at[src_v.at[0]],
                                      bufs[b], sems[b]).wait()
                pltpu.sync_copy(bufs[b], acc_sh.at[dst_v.at[jj]], add=True)
                pltpu.async_copy(hs_hbm.at[src_v.at[jj + _NBUF]], bufs[b],
                                 sems[b])

        for b in range(_NBUF):
            pltpu.make_async_copy(hs_hbm.at[src_v.at[0]], bufs[b],
                                  sems[b]).wait()
            pltpu.sync_copy(bufs[b],
                            acc_sh.at[dst_v.at[_NHALF - _NBUF + b]],
                            add=True)

    plsc.subcore_barrier()
    pltpu.sync_copy(acc_sh.at[pl.ds(row0, _STRIPE)],
                    out_hbm.at[cid].at[pl.ds(row0, _STRIPE)])


def _tc_pre_body(degp_ref, x_ref, w_ref, dinv_ref, hs_ref):
    deg = jnp.sum(degp_ref[...], axis=0).reshape(_NPAD, 1) + 1.0
    rows = lax.broadcasted_iota(jnp.int32, (_NPAD, 1), 0)
    dinv = jnp.where(rows < _N, lax.rsqrt(deg), 0.0)
    dinv_ref[...] = dinv
    h = jnp.dot(x_ref[...], w_ref[...], preferred_element_type=jnp.float32)
    hs_ref[0:_N, :] = dinv[0:_N, :] * h
    hs_ref[_N:_NPAD, :] = jnp.zeros((_NPAD - _N, _F), jnp.float32)


_tc_pre = pl.pallas_call(
    _tc_pre_body,
    out_shape=(jax.ShapeDtypeStruct((_NPAD, 1), jnp.float32),
               jax.ShapeDtypeStruct((_NPAD, _F), jnp.float32)))


def _tc_mid_body(p_ref, hs_ref, dinv_ref, b_ref, g_ref, be_ref, w_ref,
                 out_ref):
    dinv = dinv_ref[0:_N, :]
    agg = p_ref[0, 0:_N, :] + p_ref[1, 0:_N, :] + hs_ref[0:_N, :]
    pre = dinv * agg + b_ref[...]
    act = jnp.where(pre > 0, pre, 0.01 * pre)
    mu = jnp.mean(act, axis=0, keepdims=True)
    cen = act - mu
    var = jnp.mean(cen * cen, axis=0, keepdims=True)
    bn = cen * (g_ref[...] * lax.rsqrt(var + _EPS)) + be_ref[...]
    h = jnp.dot(bn, w_ref[...], preferred_element_type=jnp.float32)
    out_ref[0:_N, :] = dinv * h
    out_ref[_N:_NPAD, :] = jnp.zeros((_NPAD - _N, _F), jnp.float32)


_tc_mid = pl.pallas_call(
    _tc_mid_body,
    out_shape=jax.ShapeDtypeStruct((_NPAD, _F), jnp.float32))


def _tc_fin_body(p_ref, hs_ref, dinv_ref, b_ref, g_ref, be_ref, batch_ref,
                 out_ref):
    dinv = dinv_ref[0:_N, :]
    agg = p_ref[0, 0:_N, :] + p_ref[1, 0:_N, :] + hs_ref[0:_N, :]
    pre = dinv * agg + b_ref[...]
    act = jnp.where(pre > 0, pre, 0.01 * pre)
    mu = jnp.mean(act, axis=0, keepdims=True)
    cen = act - mu
    var = jnp.mean(cen * cen, axis=0, keepdims=True)
    bn = cen * (g_ref[...] * lax.rsqrt(var + _EPS)) + be_ref[...]
    seg = lax.broadcasted_iota(jnp.int32, (_G, _N), 0)
    onehot = (batch_ref[...].reshape(1, _N) == seg).astype(jnp.float32)
    sums = jnp.dot(onehot, bn, preferred_element_type=jnp.float32)
    cnt = jnp.sum(onehot, axis=1, keepdims=True)
    out_ref[...] = sums / jnp.maximum(cnt, 1.0)


_tc_fin = pl.pallas_call(
    _tc_fin_body,
    out_shape=jax.ShapeDtypeStruct((_G, _F), jnp.float32))


def kernel(x, W1, b1, g1, be1, W2, b2, g2, be2, W3, b3, g3, be3,
           edge_index, batch):
    src = edge_index[0].astype(jnp.int32)
    dst = edge_index[1].astype(jnp.int32)
    perw = _E // _NW
    padw = _EPT - perw
    pad = _N + (jnp.arange(_NW * padw, dtype=jnp.int32) % (_NPAD - _N))
    pad = pad.reshape(_NW, padw)
    srcp = jnp.concatenate([src.reshape(_NW, perw), pad],
                           axis=1).reshape(_NW, _NCHUNK, _CH)
    dstp = jnp.concatenate([dst.reshape(_NW, perw), pad],
                           axis=1).reshape(_NW, _NCHUNK, _CH)
    zeros = jnp.zeros((_NPAD, _F), jnp.float32)

    batchp = batch.astype(jnp.int32).reshape(1, _N)

    degp = _sc_degree(dstp)
    dinv, hs = _tc_pre(degp, x, W1)
    p = _sc_scatter(hs, srcp, dstp, zeros)
    hs = _tc_mid(p, hs, dinv, b1.reshape(1, _F), g1.reshape(1, _F),
                 be1.reshape(1, _F), W2)
    p = _sc_scatter(hs, srcp, dstp, zeros)
    hs = _tc_mid(p, hs, dinv, b2.reshape(1, _F), g2.reshape(1, _F),
                 be2.reshape(1, _F), W3)
    p = _sc_scatter(hs, srcp, dstp, zeros)
    return _tc_fin(p, hs, dinv, b3.reshape(1, _F), g3.reshape(1, _F),
                   be3.reshape(1, _F), batchp)
